# baseline (speedup 1.0000x reference)
import functools

import jax
import jax.numpy as jnp
from jax import lax
from jax.experimental import pallas as pl
from jax.experimental.pallas import tpu as pltpu
from jax.experimental.pallas import tpu_sc as plsc

S, D, H, E, K = 2048, 768, 2048, 8, 2
TBLK = 128
NPAD = S * K + E * TBLK
NT = NPAD // TBLK
NC, NSUB, LANES = 2, 16, 16
NW = NC * NSUB


def _router_body(x_ref, wr_ref, p0_ref, p1_ref, w0_ref, w1_ref, te_ref):
    xt = x_ref[...]
    logits = jnp.dot(xt, wr_ref[...], preferred_element_type=jnp.float32)
    lane = lax.broadcasted_iota(jnp.int32, (S, E), 1)
    m0 = jnp.max(logits, axis=1, keepdims=True)
    a0 = jnp.min(jnp.where(logits == m0, lane, E), axis=1, keepdims=True)
    l1 = jnp.where(lane == a0, -jnp.inf, logits)
    m1 = jnp.max(l1, axis=1, keepdims=True)
    a1 = jnp.min(jnp.where(l1 == m1, lane, E), axis=1, keepdims=True)
    wt0 = 1.0 / (1.0 + jnp.exp(m1 - m0))
    oh0 = (lane == a0).astype(jnp.float32)
    oh1 = (lane == a1).astype(jnp.float32)
    r_i = lax.broadcasted_iota(jnp.int32, (S, S), 0)
    c_i = lax.broadcasted_iota(jnp.int32, (S, S), 1)
    tri = (c_i <= r_i).astype(jnp.float32)
    cum0 = jnp.dot(tri, oh0, preferred_element_type=jnp.float32)
    cum1 = jnp.dot(tri, oh1, preferred_element_type=jnp.float32)
    cnt0 = cum0[S - 1:S, :]
    cnt1 = cum1[S - 1:S, :]
    padded = jnp.ceil((cnt0 + cnt1) / TBLK) * TBLK
    ue = lax.broadcasted_iota(jnp.int32, (E, E), 0)
    ve = lax.broadcasted_iota(jnp.int32, (E, E), 1)
    strict = (ue < ve).astype(jnp.float32)
    off = jnp.dot(padded, strict, preferred_element_type=jnp.float32)
    p0 = jnp.sum(oh0 * (off + cum0), axis=1, keepdims=True) - 1.0
    p1 = jnp.sum(oh1 * (off + cnt0 + cum1), axis=1, keepdims=True) - 1.0
    p0_ref[...] = p0.astype(jnp.int32)
    p1_ref[...] = p1.astype(jnp.int32)
    w0_ref[...] = wt0
    w1_ref[...] = 1.0 - wt0
    jt = (lax.broadcasted_iota(jnp.int32, (NT, E), 0) * TBLK).astype(jnp.float32)
    te = jnp.sum((off <= jt).astype(jnp.int32), axis=1, keepdims=True) - 1
    te_ref[...] = te


def _router(flat, Wr):
    return pl.pallas_call(
        _router_body,
        out_shape=(
            jax.ShapeDtypeStruct((S, 1), jnp.int32),
            jax.ShapeDtypeStruct((S, 1), jnp.int32),
            jax.ShapeDtypeStruct((S, 1), jnp.float32),
            jax.ShapeDtypeStruct((S, 1), jnp.float32),
            jax.ShapeDtypeStruct((NT, 1), jnp.int32),
        ),
    )(flat, Wr)


_GCH = (NPAD // NW) // 2
_CM = S // NW


@functools.lru_cache(maxsize=1)
def _sc_kernels():
    mesh = plsc.VectorSubcoreMesh(core_axis_name="c", subcore_axis_name="s")

    @functools.partial(
        pl.kernel,
        mesh=mesh,
        compiler_params=pltpu.CompilerParams(needs_layout_passes=False),
        out_type=jax.ShapeDtypeStruct((NPAD, D), jnp.float32),
        scratch_types=[
            pltpu.VMEM((_CM,), jnp.int32),
            pltpu.VMEM((_CM,), jnp.int32),
            pltpu.VMEM((_CM, D), jnp.float32),
            pltpu.SemaphoreType.DMA,
            pltpu.SemaphoreType.DMA,
        ],
    )
    def _scatter_dispatch(p0_hbm, p1_hbm, x_hbm, xg_hbm, i0, i1, xbuf, s0, s1):
        wid = lax.axis_index("s") * NC + lax.axis_index("c")
        base = wid * _CM
        pltpu.sync_copy(p0_hbm.at[pl.ds(base, _CM)], i0)
        pltpu.sync_copy(p1_hbm.at[pl.ds(base, _CM)], i1)
        pltpu.sync_copy(x_hbm.at[pl.ds(base, _CM)], xbuf)
        c0 = pltpu.async_copy(xbuf, xg_hbm.at[i0], s0)
        c1 = pltpu.async_copy(xbuf, xg_hbm.at[i1], s1)
        c0.wait()
        c1.wait()


    @functools.partial(
        pl.kernel,
        mesh=mesh,
        compiler_params=pltpu.CompilerParams(needs_layout_passes=False),
        out_type=jax.ShapeDtypeStruct((S, D), jnp.float32),
        scratch_types=[
            pltpu.VMEM((_CM,), jnp.int32),
            pltpu.VMEM((_CM,), jnp.int32),
            pltpu.VMEM((_CM,), jnp.float32),
            pltpu.VMEM((_CM,), jnp.float32),
            pltpu.VMEM((_CM, D), jnp.float32),
            pltpu.VMEM((_CM, D), jnp.float32),
            pltpu.SemaphoreType.DMA,
            pltpu.SemaphoreType.DMA,
        ],
    )
    def _combine(p0_hbm, p1_hbm, w0_hbm, w1_hbm, y_hbm, out_hbm,
                 i0, i1, g0, g1, b0, b1, s0, s1):
        wid = lax.axis_index("s") * NC + lax.axis_index("c")
        base = wid * _CM
        pltpu.sync_copy(p0_hbm.at[pl.ds(base, _CM)], i0)
        pltpu.sync_copy(p1_hbm.at[pl.ds(base, _CM)], i1)
        pltpu.sync_copy(w0_hbm.at[pl.ds(base, _CM)], g0)
        pltpu.sync_copy(w1_hbm.at[pl.ds(base, _CM)], g1)
        cp0 = pltpu.async_copy(y_hbm.at[i0], b0, s0)
        cp1 = pltpu.async_copy(y_hbm.at[i1], b1, s1)
        cp0.wait()
        cp1.wait()

        def row(r, carry):
            rr = jnp.zeros((LANES,), jnp.int32) + r
            sc0 = plsc.load_gather(g0, [rr])
            sc1 = plsc.load_gather(g1, [rr])
            for cc in range(D // LANES):
                sl = pl.ds(cc * LANES, LANES)
                b0[r, sl] = b0[r, sl] * sc0 + b1[r, sl] * sc1
            return carry

        lax.fori_loop(0, _CM, row, 0)
        pltpu.sync_copy(b0, out_hbm.at[pl.ds(base, _CM)])

    return _scatter_dispatch, _combine


NH = 4
HC = H // NH


def _ffn_body(te_ref, xg_ref, w1_ref, w3_ref, w2_ref, o_ref):
    c = pl.program_id(1)
    xt = xg_ref[...]
    h = jnp.dot(xt, w1_ref[0], preferred_element_type=jnp.float32)
    g = jnp.dot(xt, w3_ref[0], preferred_element_type=jnp.float32)
    act = h * lax.logistic(h) * g
    part = jnp.dot(act, w2_ref[0], preferred_element_type=jnp.float32)

    @pl.when(c == 0)
    def _():
        o_ref[...] = part

    @pl.when(c > 0)
    def _():
        o_ref[...] += part


def _ffn(te, xg, W1, W3, W2):
    grid_spec = pltpu.PrefetchScalarGridSpec(
        num_scalar_prefetch=1,
        grid=(NT, NH),
        in_specs=[
            pl.BlockSpec((TBLK, D), lambda j, c, te: (j, 0)),
            pl.BlockSpec((1, D, HC), lambda j, c, te: (te[j], 0, c)),
            pl.BlockSpec((1, D, HC), lambda j, c, te: (te[j], 0, c)),
            pl.BlockSpec((1, HC, D), lambda j, c, te: (te[j], c, 0)),
        ],
        out_specs=pl.BlockSpec((TBLK, D), lambda j, c, te: (j, 0)),
    )
    return pl.pallas_call(
        _ffn_body,
        grid_spec=grid_spec,
        out_shape=jax.ShapeDtypeStruct((NPAD, D), jnp.float32),
    )(te, xg, W1, W3, W2)


def kernel(x, Wr, W1, W3, W2):
    scatter_dispatch, combine = _sc_kernels()
    flat = x.reshape(S, D)
    p0, p1, w0, w1, te = _router(flat, Wr)
    p0 = p0.reshape(S)
    p1 = p1.reshape(S)
    w0 = w0.reshape(S)
    w1 = w1.reshape(S)
    te = te.reshape(NT)
    xg = scatter_dispatch(p0, p1, flat)
    y = _ffn(te, xg, W1, W3, W2)
    out = combine(p0, p1, w0, w1, y)
    return out.reshape(1, S, D)


# speedup vs baseline: 1.2181x; 1.2181x over previous
import functools

import jax
import jax.numpy as jnp
from jax import lax
from jax.experimental import pallas as pl
from jax.experimental.pallas import tpu as pltpu
from jax.experimental.pallas import tpu_sc as plsc

S, D, H, E, K = 2048, 768, 2048, 8, 2
TBLK = 128
NPAD = S * K + E * TBLK
NT = NPAD // TBLK
NC, NSUB, LANES = 2, 16, 16
NW = NC * NSUB


def _router_body(x_ref, wr_ref, p0_ref, p1_ref, w0_ref, w1_ref, te_ref):
    xt = x_ref[...]
    logits = jnp.dot(xt, wr_ref[...], preferred_element_type=jnp.float32)
    lane = lax.broadcasted_iota(jnp.int32, (S, E), 1)
    m0 = jnp.max(logits, axis=1, keepdims=True)
    a0 = jnp.min(jnp.where(logits == m0, lane, E), axis=1, keepdims=True)
    l1 = jnp.where(lane == a0, -jnp.inf, logits)
    m1 = jnp.max(l1, axis=1, keepdims=True)
    a1 = jnp.min(jnp.where(l1 == m1, lane, E), axis=1, keepdims=True)
    wt0 = 1.0 / (1.0 + jnp.exp(m1 - m0))
    oh0 = (lane == a0).astype(jnp.float32)
    oh1 = (lane == a1).astype(jnp.float32)
    r_i = lax.broadcasted_iota(jnp.int32, (S, S), 0)
    c_i = lax.broadcasted_iota(jnp.int32, (S, S), 1)
    tri = (c_i <= r_i).astype(jnp.float32)
    cum0 = jnp.dot(tri, oh0, preferred_element_type=jnp.float32)
    cum1 = jnp.dot(tri, oh1, preferred_element_type=jnp.float32)
    cnt0 = cum0[S - 1:S, :]
    cnt1 = cum1[S - 1:S, :]
    padded = jnp.ceil((cnt0 + cnt1) / TBLK) * TBLK
    ue = lax.broadcasted_iota(jnp.int32, (E, E), 0)
    ve = lax.broadcasted_iota(jnp.int32, (E, E), 1)
    strict = (ue < ve).astype(jnp.float32)
    off = jnp.dot(padded, strict, preferred_element_type=jnp.float32)
    p0 = jnp.sum(oh0 * (off + cum0), axis=1, keepdims=True) - 1.0
    p1 = jnp.sum(oh1 * (off + cnt0 + cum1), axis=1, keepdims=True) - 1.0
    p0_ref[...] = p0.astype(jnp.int32)
    p1_ref[...] = p1.astype(jnp.int32)
    w0_ref[...] = wt0
    w1_ref[...] = 1.0 - wt0
    jt = (lax.broadcasted_iota(jnp.int32, (NT, E), 0) * TBLK).astype(jnp.float32)
    te = jnp.sum((off <= jt).astype(jnp.int32), axis=1, keepdims=True) - 1
    te_ref[...] = te


def _router(flat, Wr):
    return pl.pallas_call(
        _router_body,
        out_shape=(
            jax.ShapeDtypeStruct((S, 1), jnp.int32),
            jax.ShapeDtypeStruct((S, 1), jnp.int32),
            jax.ShapeDtypeStruct((S, 1), jnp.float32),
            jax.ShapeDtypeStruct((S, 1), jnp.float32),
            jax.ShapeDtypeStruct((NT, 1), jnp.int32),
        ),
    )(flat, Wr)


_GCH = (NPAD // NW) // 2
_CM = S // NW


@functools.lru_cache(maxsize=1)
def _sc_kernels():
    mesh = plsc.VectorSubcoreMesh(core_axis_name="c", subcore_axis_name="s")

    @functools.partial(
        pl.kernel,
        mesh=mesh,
        compiler_params=pltpu.CompilerParams(needs_layout_passes=False),
        out_type=jax.ShapeDtypeStruct((NPAD, D), jnp.float32),
        scratch_types=[
            pltpu.VMEM((_CM,), jnp.int32),
            pltpu.VMEM((_CM,), jnp.int32),
            pltpu.VMEM((_CM, D), jnp.float32),
            pltpu.SemaphoreType.DMA,
            pltpu.SemaphoreType.DMA,
        ],
    )
    def _scatter_dispatch(p0_hbm, p1_hbm, x_hbm, xg_hbm, i0, i1, xbuf, s0, s1):
        wid = lax.axis_index("s") * NC + lax.axis_index("c")
        base = wid * _CM
        pltpu.sync_copy(p0_hbm.at[pl.ds(base, _CM)], i0)
        pltpu.sync_copy(p1_hbm.at[pl.ds(base, _CM)], i1)
        pltpu.sync_copy(x_hbm.at[pl.ds(base, _CM)], xbuf)
        c0 = pltpu.async_copy(xbuf, xg_hbm.at[i0], s0)
        c1 = pltpu.async_copy(xbuf, xg_hbm.at[i1], s1)
        c0.wait()
        c1.wait()


    @functools.partial(
        pl.kernel,
        mesh=mesh,
        compiler_params=pltpu.CompilerParams(needs_layout_passes=False),
        out_type=jax.ShapeDtypeStruct((S, D), jnp.float32),
        scratch_types=[
            pltpu.VMEM((_CM,), jnp.int32),
            pltpu.VMEM((_CM,), jnp.int32),
            pltpu.VMEM((_CM,), jnp.float32),
            pltpu.VMEM((_CM,), jnp.float32),
            pltpu.VMEM((_CM, D), jnp.float32),
            pltpu.VMEM((_CM, D), jnp.float32),
            pltpu.SemaphoreType.DMA,
            pltpu.SemaphoreType.DMA,
        ],
    )
    def _combine(p0_hbm, p1_hbm, w0_hbm, w1_hbm, y_hbm, out_hbm,
                 i0, i1, g0, g1, b0, b1, s0, s1):
        wid = lax.axis_index("s") * NC + lax.axis_index("c")
        base = wid * _CM
        pltpu.sync_copy(p0_hbm.at[pl.ds(base, _CM)], i0)
        pltpu.sync_copy(p1_hbm.at[pl.ds(base, _CM)], i1)
        pltpu.sync_copy(w0_hbm.at[pl.ds(base, _CM)], g0)
        pltpu.sync_copy(w1_hbm.at[pl.ds(base, _CM)], g1)
        cp0 = pltpu.async_copy(y_hbm.at[i0], b0, s0)
        cp1 = pltpu.async_copy(y_hbm.at[i1], b1, s1)
        cp0.wait()
        cp1.wait()

        def row(r, carry):
            rr = jnp.zeros((LANES,), jnp.int32) + r
            sc0 = plsc.load_gather(g0, [rr])
            sc1 = plsc.load_gather(g1, [rr])
            for cc in range(D // LANES):
                sl = pl.ds(cc * LANES, LANES)
                b0[r, sl] = b0[r, sl] * sc0 + b1[r, sl] * sc1
            return carry

        lax.fori_loop(0, _CM, row, 0)
        pltpu.sync_copy(b0, out_hbm.at[pl.ds(base, _CM)])

    return _scatter_dispatch, _combine


NH = 4
HC = H // NH


def _ffn_body(te_ref, xg_ref, w1_ref, w3_ref, w2_ref, o_ref):
    c = pl.program_id(1)
    xt = xg_ref[...]
    h = jnp.dot(xt, w1_ref[0, :, pl.ds(c * HC, HC)],
                preferred_element_type=jnp.float32)
    g = jnp.dot(xt, w3_ref[0, :, pl.ds(c * HC, HC)],
                preferred_element_type=jnp.float32)
    act = h * lax.logistic(h) * g
    part = jnp.dot(act, w2_ref[0], preferred_element_type=jnp.float32)

    @pl.when(c == 0)
    def _():
        o_ref[...] = part

    @pl.when(c > 0)
    def _():
        o_ref[...] += part


def _ffn(te, xg, W1, W3, W2):
    grid_spec = pltpu.PrefetchScalarGridSpec(
        num_scalar_prefetch=1,
        grid=(NT, NH),
        in_specs=[
            pl.BlockSpec((TBLK, D), lambda j, c, te: (j, 0)),
            pl.BlockSpec((1, D, H), lambda j, c, te: (te[j], 0, 0)),
            pl.BlockSpec((1, D, H), lambda j, c, te: (te[j], 0, 0)),
            pl.BlockSpec((1, HC, D), lambda j, c, te: (te[j], c, 0)),
        ],
        out_specs=pl.BlockSpec((TBLK, D), lambda j, c, te: (j, 0)),
    )
    return pl.pallas_call(
        _ffn_body,
        grid_spec=grid_spec,
        out_shape=jax.ShapeDtypeStruct((NPAD, D), jnp.float32),
    )(te, xg, W1, W3, W2)


def kernel(x, Wr, W1, W3, W2):
    scatter_dispatch, combine = _sc_kernels()
    flat = x.reshape(S, D)
    p0, p1, w0, w1, te = _router(flat, Wr)
    p0 = p0.reshape(S)
    p1 = p1.reshape(S)
    w0 = w0.reshape(S)
    w1 = w1.reshape(S)
    te = te.reshape(NT)
    xg = scatter_dispatch(p0, p1, flat)
    y = _ffn(te, xg, W1, W3, W2)
    out = combine(p0, p1, w0, w1, y)
    return out.reshape(1, S, D)


# speedup vs baseline: 2.1881x; 1.7963x over previous
import functools

import jax
import jax.numpy as jnp
from jax import lax
from jax.experimental import pallas as pl
from jax.experimental.pallas import tpu as pltpu
from jax.experimental.pallas import tpu_sc as plsc

S, D, H, E, K = 2048, 768, 2048, 8, 2
TBLK = 512
NPAD = S * K + E * TBLK
NT = NPAD // TBLK
NC, NSUB, LANES = 2, 16, 16
NW = NC * NSUB


def _router_body(x_ref, wr_ref, p0_ref, p1_ref, w0_ref, w1_ref, te_ref):
    xt = x_ref[...]
    logits = jnp.dot(xt, wr_ref[...], preferred_element_type=jnp.float32)
    lane = lax.broadcasted_iota(jnp.int32, (S, E), 1)
    m0 = jnp.max(logits, axis=1, keepdims=True)
    a0 = jnp.min(jnp.where(logits == m0, lane, E), axis=1, keepdims=True)
    l1 = jnp.where(lane == a0, -jnp.inf, logits)
    m1 = jnp.max(l1, axis=1, keepdims=True)
    a1 = jnp.min(jnp.where(l1 == m1, lane, E), axis=1, keepdims=True)
    wt0 = 1.0 / (1.0 + jnp.exp(m1 - m0))
    oh0 = (lane == a0).astype(jnp.float32)
    oh1 = (lane == a1).astype(jnp.float32)
    r_i = lax.broadcasted_iota(jnp.int32, (S, S), 0)
    c_i = lax.broadcasted_iota(jnp.int32, (S, S), 1)
    tri = (c_i <= r_i).astype(jnp.float32)
    cum0 = jnp.dot(tri, oh0, preferred_element_type=jnp.float32)
    cum1 = jnp.dot(tri, oh1, preferred_element_type=jnp.float32)
    cnt0 = cum0[S - 1:S, :]
    cnt1 = cum1[S - 1:S, :]
    padded = jnp.ceil((cnt0 + cnt1) / TBLK) * TBLK
    ue = lax.broadcasted_iota(jnp.int32, (E, E), 0)
    ve = lax.broadcasted_iota(jnp.int32, (E, E), 1)
    strict = (ue < ve).astype(jnp.float32)
    off = jnp.dot(padded, strict, preferred_element_type=jnp.float32)
    p0 = jnp.sum(oh0 * (off + cum0), axis=1, keepdims=True) - 1.0
    p1 = jnp.sum(oh1 * (off + cnt0 + cum1), axis=1, keepdims=True) - 1.0
    p0_ref[...] = p0.astype(jnp.int32)
    p1_ref[...] = p1.astype(jnp.int32)
    w0_ref[...] = wt0
    w1_ref[...] = 1.0 - wt0
    jt = (lax.broadcasted_iota(jnp.int32, (NT, E), 0) * TBLK).astype(jnp.float32)
    te = jnp.sum((off <= jt).astype(jnp.int32), axis=1, keepdims=True) - 1
    used = ((off[0:1, E - 1:E] + padded[0:1, E - 1:E]) / TBLK).astype(jnp.int32)
    te_ref[...] = jnp.concatenate([te, used], axis=0)


def _router(flat, Wr):
    return pl.pallas_call(
        _router_body,
        out_shape=(
            jax.ShapeDtypeStruct((S, 1), jnp.int32),
            jax.ShapeDtypeStruct((S, 1), jnp.int32),
            jax.ShapeDtypeStruct((S, 1), jnp.float32),
            jax.ShapeDtypeStruct((S, 1), jnp.float32),
            jax.ShapeDtypeStruct((NT + 1, 1), jnp.int32),
        ),
    )(flat, Wr)


_GCH = (NPAD // NW) // 2
_CM = S // NW


@functools.lru_cache(maxsize=1)
def _sc_kernels():
    mesh = plsc.VectorSubcoreMesh(core_axis_name="c", subcore_axis_name="s")

    @functools.partial(
        pl.kernel,
        mesh=mesh,
        compiler_params=pltpu.CompilerParams(needs_layout_passes=False),
        out_type=jax.ShapeDtypeStruct((NPAD, D), jnp.float32),
        scratch_types=[
            pltpu.VMEM((_CM,), jnp.int32),
            pltpu.VMEM((_CM,), jnp.int32),
            pltpu.VMEM((_CM, D), jnp.float32),
            pltpu.SemaphoreType.DMA,
            pltpu.SemaphoreType.DMA,
        ],
    )
    def _scatter_dispatch(p0_hbm, p1_hbm, x_hbm, xg_hbm, i0, i1, xbuf, s0, s1):
        wid = lax.axis_index("s") * NC + lax.axis_index("c")
        base = wid * _CM
        pltpu.sync_copy(p0_hbm.at[pl.ds(base, _CM)], i0)
        pltpu.sync_copy(p1_hbm.at[pl.ds(base, _CM)], i1)
        pltpu.sync_copy(x_hbm.at[pl.ds(base, _CM)], xbuf)
        c0 = pltpu.async_copy(xbuf, xg_hbm.at[i0], s0)
        c1 = pltpu.async_copy(xbuf, xg_hbm.at[i1], s1)
        c0.wait()
        c1.wait()


    @functools.partial(
        pl.kernel,
        mesh=mesh,
        compiler_params=pltpu.CompilerParams(needs_layout_passes=False),
        out_type=jax.ShapeDtypeStruct((S, D), jnp.float32),
        scratch_types=[
            pltpu.VMEM((_CM,), jnp.int32),
            pltpu.VMEM((_CM,), jnp.int32),
            pltpu.VMEM((_CM,), jnp.float32),
            pltpu.VMEM((_CM,), jnp.float32),
            pltpu.VMEM((_CM, D), jnp.float32),
            pltpu.VMEM((_CM, D), jnp.float32),
            pltpu.SemaphoreType.DMA,
            pltpu.SemaphoreType.DMA,
        ],
    )
    def _combine(p0_hbm, p1_hbm, w0_hbm, w1_hbm, y_hbm, out_hbm,
                 i0, i1, g0, g1, b0, b1, s0, s1):
        wid = lax.axis_index("s") * NC + lax.axis_index("c")
        base = wid * _CM
        pltpu.sync_copy(p0_hbm.at[pl.ds(base, _CM)], i0)
        pltpu.sync_copy(p1_hbm.at[pl.ds(base, _CM)], i1)
        pltpu.sync_copy(w0_hbm.at[pl.ds(base, _CM)], g0)
        pltpu.sync_copy(w1_hbm.at[pl.ds(base, _CM)], g1)
        cp0 = pltpu.async_copy(y_hbm.at[i0], b0, s0)
        cp1 = pltpu.async_copy(y_hbm.at[i1], b1, s1)
        cp0.wait()
        cp1.wait()

        def row(r, carry):
            rr = jnp.zeros((LANES,), jnp.int32) + r
            sc0 = plsc.load_gather(g0, [rr])
            sc1 = plsc.load_gather(g1, [rr])
            for cc in range(D // LANES):
                sl = pl.ds(cc * LANES, LANES)
                b0[r, sl] = b0[r, sl] * sc0 + b1[r, sl] * sc1
            return carry

        lax.fori_loop(0, _CM, row, 0)
        pltpu.sync_copy(b0, out_hbm.at[pl.ds(base, _CM)])

    return _scatter_dispatch, _combine


def _ffn_body(te_ref, xg_ref, w1_ref, w3_ref, w2_ref, o_ref):
    j = pl.program_id(0)

    @pl.when(j < te_ref[NT])
    def _():
        xt = xg_ref[...]
        h = jnp.dot(xt, w1_ref[0], preferred_element_type=jnp.float32)
        g = jnp.dot(xt, w3_ref[0], preferred_element_type=jnp.float32)
        act = h * lax.logistic(h) * g
        o_ref[...] = jnp.dot(act, w2_ref[0], preferred_element_type=jnp.float32)


def _ffn(te, xg, W1, W3, W2):
    grid_spec = pltpu.PrefetchScalarGridSpec(
        num_scalar_prefetch=1,
        grid=(NT,),
        in_specs=[
            pl.BlockSpec((TBLK, D), lambda j, te: (j, 0)),
            pl.BlockSpec((1, D, H), lambda j, te: (te[j], 0, 0)),
            pl.BlockSpec((1, D, H), lambda j, te: (te[j], 0, 0)),
            pl.BlockSpec((1, H, D), lambda j, te: (te[j], 0, 0)),
        ],
        out_specs=pl.BlockSpec((TBLK, D), lambda j, te: (j, 0)),
    )
    return pl.pallas_call(
        _ffn_body,
        grid_spec=grid_spec,
        out_shape=jax.ShapeDtypeStruct((NPAD, D), jnp.float32),
    )(te, xg, W1, W3, W2)


def kernel(x, Wr, W1, W3, W2):
    scatter_dispatch, combine = _sc_kernels()
    flat = x.reshape(S, D)
    p0, p1, w0, w1, te = _router(flat, Wr)
    p0 = p0.reshape(S)
    p1 = p1.reshape(S)
    w0 = w0.reshape(S)
    w1 = w1.reshape(S)
    te = te.reshape(NT + 1)
    xg = scatter_dispatch(p0, p1, flat)
    y = _ffn(te, xg, W1, W3, W2)
    out = combine(p0, p1, w0, w1, y)
    return out.reshape(1, S, D)


# speedup vs baseline: 2.1934x; 1.0024x over previous
import functools

import jax
import jax.numpy as jnp
from jax import lax
from jax.experimental import pallas as pl
from jax.experimental.pallas import tpu as pltpu
from jax.experimental.pallas import tpu_sc as plsc

S, D, H, E, K = 2048, 768, 2048, 8, 2
TBLK = 512
NPAD = S * K + E * TBLK
NT = NPAD // TBLK
NC, NSUB, LANES = 2, 16, 16
NW = NC * NSUB


def _router_body(x_ref, wr_ref, p0_ref, p1_ref, w0_ref, w1_ref, te_ref):
    xt = x_ref[...]
    logits = jnp.dot(xt, wr_ref[...], preferred_element_type=jnp.float32)
    lane = lax.broadcasted_iota(jnp.int32, (S, E), 1)
    m0 = jnp.max(logits, axis=1, keepdims=True)
    a0 = jnp.min(jnp.where(logits == m0, lane, E), axis=1, keepdims=True)
    l1 = jnp.where(lane == a0, -jnp.inf, logits)
    m1 = jnp.max(l1, axis=1, keepdims=True)
    a1 = jnp.min(jnp.where(l1 == m1, lane, E), axis=1, keepdims=True)
    wt0 = 1.0 / (1.0 + jnp.exp(m1 - m0))
    oh0 = (lane == a0).astype(jnp.float32)
    oh1 = (lane == a1).astype(jnp.float32)
    r_i = lax.broadcasted_iota(jnp.int32, (S, S), 0)
    c_i = lax.broadcasted_iota(jnp.int32, (S, S), 1)
    tri = (c_i <= r_i).astype(jnp.float32)
    cum0 = jnp.dot(tri, oh0, preferred_element_type=jnp.float32)
    cum1 = jnp.dot(tri, oh1, preferred_element_type=jnp.float32)
    cnt0 = cum0[S - 1:S, :]
    cnt1 = cum1[S - 1:S, :]
    padded = jnp.ceil((cnt0 + cnt1) / TBLK) * TBLK
    ue = lax.broadcasted_iota(jnp.int32, (E, E), 0)
    ve = lax.broadcasted_iota(jnp.int32, (E, E), 1)
    strict = (ue < ve).astype(jnp.float32)
    off = jnp.dot(padded, strict, preferred_element_type=jnp.float32)
    p0 = jnp.sum(oh0 * (off + cum0), axis=1, keepdims=True) - 1.0
    p1 = jnp.sum(oh1 * (off + cnt0 + cum1), axis=1, keepdims=True) - 1.0
    p0_ref[...] = p0.astype(jnp.int32)
    p1_ref[...] = p1.astype(jnp.int32)
    w0_ref[...] = wt0
    w1_ref[...] = 1.0 - wt0
    jt = (lax.broadcasted_iota(jnp.int32, (NT, E), 0) * TBLK).astype(jnp.float32)
    te = jnp.sum((off <= jt).astype(jnp.int32), axis=1, keepdims=True) - 1
    used = ((off[0:1, E - 1:E] + padded[0:1, E - 1:E]) / TBLK).astype(jnp.int32)
    te_ref[...] = jnp.concatenate([te, used], axis=0)


def _router(flat, Wr):
    return pl.pallas_call(
        _router_body,
        out_shape=(
            jax.ShapeDtypeStruct((S, 1), jnp.int32),
            jax.ShapeDtypeStruct((S, 1), jnp.int32),
            jax.ShapeDtypeStruct((S, 1), jnp.float32),
            jax.ShapeDtypeStruct((S, 1), jnp.float32),
            jax.ShapeDtypeStruct((NT + 1, 1), jnp.int32),
        ),
    )(flat, Wr)


_GCH = (NPAD // NW) // 2
_CM = S // NW


@functools.lru_cache(maxsize=1)
def _sc_kernels():
    mesh = plsc.VectorSubcoreMesh(core_axis_name="c", subcore_axis_name="s")

    @functools.partial(
        pl.kernel,
        mesh=mesh,
        compiler_params=pltpu.CompilerParams(needs_layout_passes=False),
        out_type=jax.ShapeDtypeStruct((NPAD, D), jnp.float32),
        scratch_types=[
            pltpu.VMEM((_CM,), jnp.int32),
            pltpu.VMEM((_CM,), jnp.int32),
            pltpu.VMEM((_CM, D), jnp.float32),
            pltpu.SemaphoreType.DMA,
            pltpu.SemaphoreType.DMA,
        ],
    )
    def _scatter_dispatch(p0_hbm, p1_hbm, x_hbm, xg_hbm, i0, i1, xbuf, s0, s1):
        wid = lax.axis_index("s") * NC + lax.axis_index("c")
        base = wid * _CM
        pltpu.sync_copy(p0_hbm.at[pl.ds(base, _CM)], i0)
        pltpu.sync_copy(p1_hbm.at[pl.ds(base, _CM)], i1)
        pltpu.sync_copy(x_hbm.at[pl.ds(base, _CM)], xbuf)
        c0 = pltpu.async_copy(xbuf, xg_hbm.at[i0], s0)
        c1 = pltpu.async_copy(xbuf, xg_hbm.at[i1], s1)
        c0.wait()
        c1.wait()


    @functools.partial(
        pl.kernel,
        mesh=mesh,
        compiler_params=pltpu.CompilerParams(needs_layout_passes=False),
        out_type=jax.ShapeDtypeStruct((S, D), jnp.float32),
        scratch_types=[
            pltpu.VMEM((_CM,), jnp.int32),
            pltpu.VMEM((_CM,), jnp.int32),
            pltpu.VMEM((_CM,), jnp.float32),
            pltpu.VMEM((_CM,), jnp.float32),
            pltpu.VMEM((_CM, D), jnp.float32),
            pltpu.VMEM((_CM, D), jnp.float32),
            pltpu.SemaphoreType.DMA,
            pltpu.SemaphoreType.DMA,
        ],
    )
    def _combine(p0_hbm, p1_hbm, w0_hbm, w1_hbm, y_hbm, out_hbm,
                 i0, i1, g0, g1, b0, b1, s0, s1):
        wid = lax.axis_index("s") * NC + lax.axis_index("c")
        base = wid * _CM
        pltpu.sync_copy(p0_hbm.at[pl.ds(base, _CM)], i0)
        pltpu.sync_copy(p1_hbm.at[pl.ds(base, _CM)], i1)
        pltpu.sync_copy(w0_hbm.at[pl.ds(base, _CM)], g0)
        pltpu.sync_copy(w1_hbm.at[pl.ds(base, _CM)], g1)
        cp0 = pltpu.async_copy(y_hbm.at[i0], b0, s0)
        cp1 = pltpu.async_copy(y_hbm.at[i1], b1, s1)
        cp0.wait()
        cp1.wait()

        def row(r, carry):
            rr = jnp.zeros((LANES,), jnp.int32) + r
            sc0 = plsc.load_gather(g0, [rr])
            sc1 = plsc.load_gather(g1, [rr])
            for cc in range(D // LANES):
                sl = pl.ds(cc * LANES, LANES)
                b0[r, sl] = b0[r, sl] * sc0 + b1[r, sl] * sc1
            return carry

        lax.fori_loop(0, _CM, row, 0)
        pltpu.sync_copy(b0, out_hbm.at[pl.ds(base, _CM)])

    return _scatter_dispatch, _combine


def _ffn_body(te_ref, xg_ref, w1_ref, w3_ref, w2_ref, o_ref):
    j = pl.program_id(0)

    @pl.when(j < te_ref[NT])
    def _():
        xt = xg_ref[...]
        h = jnp.dot(xt, w1_ref[0], preferred_element_type=jnp.float32,
                    precision=lax.Precision.DEFAULT)
        g = jnp.dot(xt, w3_ref[0], preferred_element_type=jnp.float32,
                    precision=lax.Precision.DEFAULT)
        act = h * lax.logistic(h) * g
        o_ref[...] = jnp.dot(act, w2_ref[0], preferred_element_type=jnp.float32,
                             precision=lax.Precision.DEFAULT)


def _ffn(te, xg, W1, W3, W2):
    grid_spec = pltpu.PrefetchScalarGridSpec(
        num_scalar_prefetch=1,
        grid=(NT,),
        in_specs=[
            pl.BlockSpec((TBLK, D), lambda j, te: (j, 0)),
            pl.BlockSpec((1, D, H), lambda j, te: (te[j], 0, 0)),
            pl.BlockSpec((1, D, H), lambda j, te: (te[j], 0, 0)),
            pl.BlockSpec((1, H, D), lambda j, te: (te[j], 0, 0)),
        ],
        out_specs=pl.BlockSpec((TBLK, D), lambda j, te: (j, 0)),
    )
    return pl.pallas_call(
        _ffn_body,
        grid_spec=grid_spec,
        out_shape=jax.ShapeDtypeStruct((NPAD, D), jnp.float32),
    )(te, xg, W1, W3, W2)


def kernel(x, Wr, W1, W3, W2):
    scatter_dispatch, combine = _sc_kernels()
    flat = x.reshape(S, D)
    p0, p1, w0, w1, te = _router(flat, Wr)
    p0 = p0.reshape(S)
    p1 = p1.reshape(S)
    w0 = w0.reshape(S)
    w1 = w1.reshape(S)
    te = te.reshape(NT + 1)
    xg = scatter_dispatch(p0, p1, flat)
    y = _ffn(te, xg, W1, W3, W2)
    out = combine(p0, p1, w0, w1, y)
    return out.reshape(1, S, D)
